# baseline (speedup 1.0000x reference)
import functools

import jax
import jax.numpy as jnp
from jax import lax
from jax.experimental import pallas as pl
from jax.experimental.pallas import tpu as pltpu
from jax.experimental.pallas import tpu_sc as plsc

_T = 32768
_D = 8
_H = 16
_NW = 32
_CHUNK = _T // _NW
_GROUPS = _CHUNK // 16
_NPARAM = 624
_U = 4

_INV_SQRT2 = 0.7071067811865476


def _gelu_erf(u):
    a = jnp.abs(u)
    z = a * _INV_SQRT2
    t = 1.0 / (1.0 + 0.3275911 * z)
    poly = t * (0.254829592 + t * (-0.284496736 + t * (1.421413741
               + t * (-1.453152027 + t * 1.061405429))))
    erf = 1.0 - poly * jnp.exp(-z * z)
    return 0.5 * (u + a * erf)


def _sc_body(x_hbm, p_hbm, out_hbm, *refs):
    x_vs = refs[:_D]
    p_v, acc_v = refs[_D], refs[_D + 1]
    nc = plsc.get_sparse_core_info().num_cores
    wid = lax.axis_index("s") * nc + lax.axis_index("c")
    base = wid * _CHUNK
    for k in range(_D):
        pltpu.sync_copy(x_hbm.at[k, pl.ds(base, _CHUNK)], x_vs[k])
    pltpu.sync_copy(p_hbm, p_v)


    def group(gi, acc):
        tok = gi * (16 * _U)
        xk = [[x_vs[k][pl.ds(tok + 16 * u, 16)] for k in range(_D)]
              for u in range(_U)]
        x2 = [[p_v[272 + i] for i in range(_D)] for _ in range(_U)]
        for j in range(_H):
            wj = [p_v[j * _D + k] for k in range(_D)]
            bj = p_v[128 + j]
            cj = [p_v[144 + i * _H + j] for i in range(_D)]
            for u in range(_U):
                a = xk[u][0] * wj[0]
                for k in range(1, _D):
                    a = a + xk[u][k] * wj[k]
                hj = jnp.maximum(a + bj, 0.0)
                for i in range(_D):
                    x2[u][i] = x2[u][i] + cj[i] * hj
        gv = [p_v[280 + k] for k in range(_D)]
        gb = p_v[288]
        mask = []
        for u in range(_U):
            d = x2[u][0] * gv[0]
            for k in range(1, _D):
                d = d + x2[u][k] * gv[k]
            mask.append((d + gb) >= 0.0)
        c0, c1 = p_v[609], p_v[610]
        s = [jnp.where(mask[u], c0, c1) for u in range(_U)]
        for j in range(_H):
            w0 = [p_v[289 + j * _D + k] for k in range(_D)]
            w1 = [p_v[417 + j * _D + k] for k in range(_D)]
            b0, b1 = p_v[545 + j], p_v[561 + j]
            v0, v1 = p_v[577 + j], p_v[593 + j]
            for u in range(_U):
                a0 = x2[u][0] * w0[0]
                a1 = x2[u][0] * w1[0]
                for k in range(1, _D):
                    a0 = a0 + x2[u][k] * w0[k]
                    a1 = a1 + x2[u][k] * w1[k]
                pre = jnp.where(mask[u], a0 + b0, a1 + b1)
                hh = _gelu_erf(pre)
                s[u] = s[u] + hh * jnp.where(mask[u], v0, v1)
        for u in range(_U):
            acc = acc + s[u]
        return acc

    acc = lax.fori_loop(0, _GROUPS // _U, group,
                        jnp.zeros((16,), jnp.float32))
    acc_v[...] = acc
    pltpu.sync_copy(acc_v, out_hbm.at[wid])


@jax.jit
def kernel(inp, W_nm1, b_nm1, W_nm2, b_nm2, Wg, bg, W1, b1, W2, b2):
    g = Wg[0] - Wg[1]
    gb = bg[0] - bg[1]
    v = W2.sum(axis=1)
    c = b2.sum(axis=1)
    params = jnp.concatenate([
        W_nm1.ravel(), b_nm1, W_nm2.ravel(), b_nm2, g, gb[None],
        W1.ravel(), b1.ravel(), v.ravel(), c,
    ])
    params = jnp.pad(params, (0, _NPARAM - params.shape[0]))
    params = jnp.broadcast_to(params[:, None], (_NPARAM, 16))
    xt = inp.T

    run = pl.kernel(
        _sc_body,
        out_type=jax.ShapeDtypeStruct((_NW, 16), jnp.float32),
        mesh=plsc.VectorSubcoreMesh(core_axis_name="c", subcore_axis_name="s"),
        scratch_types=(
            [pltpu.VMEM((_CHUNK,), jnp.float32) for _ in range(_D)]
            + [pltpu.VMEM((_NPARAM, 16), jnp.float32),
               pltpu.VMEM((16,), jnp.float32)]
        ),
    )
    partials = run(xt, params)
    return jnp.sum(partials)


# speedup vs baseline: 2.0289x; 2.0289x over previous
import functools

import jax
import jax.numpy as jnp
from jax import lax
from jax.experimental import pallas as pl
from jax.experimental.pallas import tpu as pltpu
from jax.experimental.pallas import tpu_sc as plsc

_T = 32768
_D = 8
_H = 16
_NW = 32
_CHUNK = _T // _NW
_GROUPS = _CHUNK // 16
_NPARAM = 624
_U = 2

_INV_SQRT2 = 0.7071067811865476


def _gelu_erf(u):
    a = jnp.abs(u)
    z = a * _INV_SQRT2
    t = 1.0 / (1.0 + 0.3275911 * z)
    poly = t * (0.254829592 + t * (-0.284496736 + t * (1.421413741
               + t * (-1.453152027 + t * 1.061405429))))
    erf = 1.0 - poly * jnp.exp(-z * z)
    return 0.5 * (u + a * erf)


def _sc_body(x_hbm, p_hbm, out_hbm, *refs):
    x_vs = refs[:_D]
    p_v, acc_v = refs[_D], refs[_D + 1]
    nc = plsc.get_sparse_core_info().num_cores
    wid = lax.axis_index("s") * nc + lax.axis_index("c")
    base = wid * _CHUNK
    for k in range(_D):
        pltpu.sync_copy(x_hbm.at[k, pl.ds(base, _CHUNK)], x_vs[k])
    pltpu.sync_copy(p_hbm, p_v)


    def group(gi, acc):
        tok = gi * (16 * _U)
        xk = [[x_vs[k][pl.ds(tok + 16 * u, 16)] for k in range(_D)]
              for u in range(_U)]
        x2 = [[p_v[272 + i] for i in range(_D)] for _ in range(_U)]
        for j in range(_H):
            wj = [p_v[j * _D + k] for k in range(_D)]
            bj = p_v[128 + j]
            cj = [p_v[144 + i * _H + j] for i in range(_D)]
            for u in range(_U):
                a = xk[u][0] * wj[0]
                for k in range(1, _D):
                    a = a + xk[u][k] * wj[k]
                hj = jnp.maximum(a + bj, 0.0)
                for i in range(_D):
                    x2[u][i] = x2[u][i] + cj[i] * hj
        gv = [p_v[280 + k] for k in range(_D)]
        gb = p_v[288]
        mask = []
        for u in range(_U):
            d = x2[u][0] * gv[0]
            for k in range(1, _D):
                d = d + x2[u][k] * gv[k]
            mask.append((d + gb) >= 0.0)
        c0, c1 = p_v[609], p_v[610]
        s = [jnp.where(mask[u], c0, c1) for u in range(_U)]
        for j in range(_H):
            w0 = [p_v[289 + j * _D + k] for k in range(_D)]
            w1 = [p_v[417 + j * _D + k] for k in range(_D)]
            b0, b1 = p_v[545 + j], p_v[561 + j]
            v0, v1 = p_v[577 + j], p_v[593 + j]
            for u in range(_U):
                a0 = x2[u][0] * w0[0]
                a1 = x2[u][0] * w1[0]
                for k in range(1, _D):
                    a0 = a0 + x2[u][k] * w0[k]
                    a1 = a1 + x2[u][k] * w1[k]
                pre = jnp.where(mask[u], a0 + b0, a1 + b1)
                hh = _gelu_erf(pre)
                s[u] = s[u] + hh * jnp.where(mask[u], v0, v1)
        for u in range(_U):
            acc = acc + s[u]
        return acc

    acc = lax.fori_loop(0, _GROUPS // _U, group,
                        jnp.zeros((16,), jnp.float32))
    acc_v[...] = acc
    pltpu.sync_copy(acc_v, out_hbm.at[wid])


@jax.jit
def kernel(inp, W_nm1, b_nm1, W_nm2, b_nm2, Wg, bg, W1, b1, W2, b2):
    g = Wg[0] - Wg[1]
    gb = bg[0] - bg[1]
    v = W2.sum(axis=1)
    c = b2.sum(axis=1)
    params = jnp.concatenate([
        W_nm1.ravel(), b_nm1, W_nm2.ravel(), b_nm2, g, gb[None],
        W1.ravel(), b1.ravel(), v.ravel(), c,
    ])
    params = jnp.pad(params, (0, _NPARAM - params.shape[0]))
    params = jnp.broadcast_to(params[:, None], (_NPARAM, 16))
    xt = inp.T

    run = pl.kernel(
        _sc_body,
        out_type=jax.ShapeDtypeStruct((_NW, 16), jnp.float32),
        mesh=plsc.VectorSubcoreMesh(core_axis_name="c", subcore_axis_name="s"),
        scratch_types=(
            [pltpu.VMEM((_CHUNK,), jnp.float32) for _ in range(_D)]
            + [pltpu.VMEM((_NPARAM, 16), jnp.float32),
               pltpu.VMEM((16,), jnp.float32)]
        ),
    )
    partials = run(xt, params)
    return jnp.sum(partials)
